# baseline (speedup 1.0000x reference)
import functools

import jax
import jax.numpy as jnp
import numpy as np
from jax.experimental import pallas as pl
from jax.experimental.pallas import tpu as pltpu

BM = 200
BK = 1024
CACHE_BLKS = 6


def _chunks(n):
    offs, widths = [], []
    o = 0
    while o < n:
        w = min(BK, n - o)
        offs.append(o)
        widths.append(w)
        o += w
    return list(zip(offs, widths))


def _s1_kernel(x_ref, w1_ref, b1_ref, s1_ref):
    s1_ref[...] = (
        jnp.dot(x_ref[...], w1_ref[...], preferred_element_type=jnp.float32)
        + b1_ref[...]
    ).astype(jnp.bfloat16)


def _gcn_kernel(rb_ref, tr_ref, tc_ref, r_ref, c_ref, cf_ref,
                s1_ref, adjr_ref, adjt_ref, w2_ref, b2_ref,
                out_ref, s2_ref, acc_ref, cache_ref,
                *, num_i, n_c, n, h_dim, c_dim):
    i = pl.program_id(0)

    @pl.when(i == 0)
    def _init():
        s2_ref[...] = jnp.zeros_like(s2_ref)

    @pl.when(i < num_i)
    def _phase1():
        b = i
        h = jnp.zeros((BM, h_dim), jnp.float32)
        a8 = jnp.zeros((BM, c_dim), jnp.float32)
        for off, w in _chunks(n):
            ch = adjr_ref[:, off:off + w].astype(jnp.bfloat16)
            h = h + jnp.dot(ch, s1_ref[off:off + w, :],
                            preferred_element_type=jnp.float32)
            a8 = a8 + jnp.dot(ch, s2_ref[off:off + w, :],
                              preferred_element_type=jnp.float32)

            @pl.when(b < CACHE_BLKS)
            def _fill_cache():
                cache_ref[pl.ds(b * BM, BM), off:off + w] = ch

        acc_ref[pl.ds(b * BM, BM), :] = a8
        s2_ref[pl.ds(b * BM, BM), :] = (
            jnp.dot(jnp.maximum(h, 0.0), w2_ref[...],
                    preferred_element_type=jnp.float32)
            + b2_ref[...]
        ).astype(jnp.bfloat16)

    @pl.when(i >= num_i)
    def _phase2():
        r = r_ref[i]
        c = c_ref[i]
        cached = cf_ref[i]

        @pl.when(cached == 1)
        def _cached_row():
            rowbf = cache_ref[pl.ds(r * BM, BM), :]
            o = jnp.dot(rowbf, s2_ref[0:n, :],
                        preferred_element_type=jnp.float32)
            m = jnp.max(o, axis=1, keepdims=True)
            lse = jnp.log(jnp.sum(jnp.exp(o - m), axis=1, keepdims=True)) + m
            out_ref[...] = o - lse

        @pl.when(cached == 0)
        def _tile():
            s2s = s2_ref[pl.ds(c * BK, BK), :]
            row_idx = jax.lax.broadcasted_iota(jnp.int32, (BK, 1), 0)
            s2m = jnp.where(c * BK + row_idx >= r * BM, s2s,
                            jnp.bfloat16(0.0))
            prev = acc_ref[pl.ds(r * BM, BM), :]

            @pl.when(c == n_c - 1)
            def _final():
                col_idx = jax.lax.broadcasted_iota(jnp.int32, (1, BK), 1)
                tile = jnp.where(c * BK + col_idx < n, adjt_ref[...], 0.0)
                tot = prev + jnp.dot(tile.astype(jnp.bfloat16), s2m,
                                     preferred_element_type=jnp.float32)
                m = jnp.max(tot, axis=1, keepdims=True)
                lse = jnp.log(jnp.sum(jnp.exp(tot - m), axis=1,
                                      keepdims=True)) + m
                out_ref[...] = tot - lse

            @pl.when(c < n_c - 1)
            def _accum():
                acc_ref[pl.ds(r * BM, BM), :] = prev + jnp.dot(
                    adjt_ref[...].astype(jnp.bfloat16), s2m,
                    preferred_element_type=jnp.float32)


def _schedule(num_i, n_c, cache_blks):
    rb, tr, tc, rr, cc, cf = [], [], [], [], [], []
    park_r, park_c = cache_blks, (cache_blks * BM) // BK
    for b in range(num_i):
        rb.append(b); tr.append(park_r); tc.append(park_c)
        rr.append(0); cc.append(0); cf.append(0)
    for r in range(cache_blks):
        rb.append(num_i - 1); tr.append(park_r); tc.append(park_c)
        rr.append(r); cc.append(n_c - 1); cf.append(1)
    for r in range(cache_blks, num_i):
        c0 = (r * BM) // BK
        for c in range(c0, n_c):
            rb.append(num_i - 1); tr.append(r); tc.append(c)
            rr.append(r); cc.append(c); cf.append(0)
    arrs = [np.asarray(a, dtype=np.int32) for a in (rb, tr, tc, rr, cc, cf)]
    return arrs


@jax.jit
def kernel(x, adj, W1, b1, W2, b2):
    n, f_in = x.shape
    h_dim = W1.shape[1]
    c_dim = W2.shape[1]
    num_i = n // BM
    n_c = -(-n // BK)
    cache_blks = min(CACHE_BLKS, num_i)

    b1r = b1.reshape(1, h_dim)
    b2r = b2.reshape(1, c_dim)

    s1 = pl.pallas_call(
        _s1_kernel,
        out_shape=jax.ShapeDtypeStruct((n, h_dim), jnp.bfloat16),
    )(x, W1, b1r)

    arrs = _schedule(num_i, n_c, cache_blks)
    t = arrs[0].shape[0]

    grid_spec = pltpu.PrefetchScalarGridSpec(
        num_scalar_prefetch=6,
        grid=(t,),
        in_specs=[
            pl.BlockSpec((n, h_dim), lambda i, *s: (0, 0)),
            pl.BlockSpec((BM, n), lambda i, *s: (s[0][i], 0)),
            pl.BlockSpec((BM, BK), lambda i, *s: (s[1][i], s[2][i])),
            pl.BlockSpec((h_dim, c_dim), lambda i, *s: (0, 0)),
            pl.BlockSpec((1, c_dim), lambda i, *s: (0, 0)),
        ],
        out_specs=pl.BlockSpec((BM, c_dim), lambda i, *s: (s[3][i], 0)),
        scratch_shapes=[
            pltpu.VMEM((n_c * BK, c_dim), jnp.bfloat16),
            pltpu.VMEM((n, c_dim), jnp.float32),
            pltpu.VMEM((cache_blks * BM, n), jnp.bfloat16),
        ],
    )

    return pl.pallas_call(
        functools.partial(_gcn_kernel, num_i=num_i, n_c=n_c, n=n,
                          h_dim=h_dim, c_dim=c_dim),
        grid_spec=grid_spec,
        out_shape=jax.ShapeDtypeStruct((n, c_dim), jnp.float32),
        compiler_params=pltpu.CompilerParams(
            dimension_semantics=("arbitrary",),
        ),
    )(*arrs, s1, adj, adj, W2, b2r)
